# baseline (speedup 1.0000x reference)
import functools

import jax
import jax.numpy as jnp
from jax import lax
from jax.experimental import pallas as pl
from jax.experimental.pallas import tpu as pltpu
from jax.experimental.pallas import tpu_sc as plsc

_THRESH = 0.055
_N = 500
_E = 128
_S = 16


def _one_sample(f, ft, w0t, w0b, w2t, w2b):
    xc = f[:, 0:1]
    yc = f[:, 1:2]
    xr = ft[0:1, :]
    yr = ft[1:2, :]
    dx = xc - xr
    dy = yc - yr
    dist2 = dx * dx + dy * dy
    A = (dist2 <= _THRESH * _THRESH).astype(jnp.float32)
    Ab = A.astype(jnp.bfloat16)

    fv0 = jnp.maximum(
        jnp.dot(f, w0t, preferred_element_type=jnp.float32) + w0b, 0.0)
    fh = fv0.astype(jnp.bfloat16)
    fl = (fv0 - fh.astype(jnp.float32)).astype(jnp.bfloat16)
    zcols = jnp.zeros((_N, 12), jnp.bfloat16)
    AF = jnp.concatenate([Ab, zcols, fh, fl], axis=1)
    R = jnp.dot(Ab, AF, preferred_element_type=jnp.float32)
    C = R[:, :_N]
    fv1 = R[:, 512:512 + _E] + R[:, 512 + _E:512 + 2 * _E]
    B2 = (C > 0).astype(jnp.float32)
    D = jnp.dot(B2.astype(jnp.bfloat16), Ab, preferred_element_type=jnp.float32)
    M = B2 * D

    g = jnp.dot(fv1, w2t, preferred_element_type=jnp.float32)
    Fv2 = jnp.maximum(
        jnp.dot(M, g, preferred_element_type=jnp.float32) + w2b, 0.0)
    return Fv2


def _ccn2_body(feat_ref, featT_ref, w0t_ref, w0b_ref, w2t_ref, w2b_ref,
               out_ref, mean_ref):
    for s in range(_S):
        Fv2 = _one_sample(feat_ref[s], featT_ref[s], w0t_ref[...],
                          w0b_ref[...], w2t_ref[...], w2b_ref[...])
        out_ref[s] = Fv2
        mean_ref[s, 0] = jnp.mean(Fv2, axis=0)


def _sc_mean(Fv2):
    B = Fv2.shape[0]
    per_w = B // 32

    mesh = plsc.VectorSubcoreMesh(core_axis_name="c", subcore_axis_name="s")

    @functools.partial(
        pl.kernel, mesh=mesh,
        out_type=jax.ShapeDtypeStruct((B, _E), jnp.float32),
        scratch_types=[
            pltpu.VMEM((_N, _E), jnp.float32),
            pltpu.VMEM((_E,), jnp.float32),
        ],
    )
    def sc_mean_kernel(fv2_hbm, out_hbm, buf, acc):
        wid = lax.axis_index("s") * 2 + lax.axis_index("c")
        for k in range(per_w):
            sample = wid * per_w + k
            pltpu.sync_copy(fv2_hbm.at[sample], buf)
            for j in range(_E // 16):
                acc[pl.ds(j * 16, 16)] = jnp.zeros((16,), jnp.float32)

            def body(i, carry):
                for j in range(_E // 16):
                    sl = pl.ds(j * 16, 16)
                    acc[sl] = acc[sl] + buf[i, sl]
                return carry

            lax.fori_loop(0, _N, body, 0)
            inv = jnp.float32(1.0 / _N)
            for j in range(_E // 16):
                sl = pl.ds(j * 16, 16)
                acc[sl] = acc[sl] * inv
            pltpu.sync_copy(acc, out_hbm.at[sample])

    return sc_mean_kernel(Fv2)


@functools.partial(jax.jit, static_argnames=())
def kernel(loc, deadline, depot, W0_w, W0_b, W2_w, W2_b):
    B = loc.shape[0]
    locations = jnp.concatenate([depot[:, None, :], loc], axis=1)
    td = jnp.concatenate(
        [jnp.zeros((B, 1), deadline.dtype), deadline], axis=1)
    feat = jnp.concatenate([locations, td[..., None]], axis=-1)
    featT = jnp.swapaxes(feat, 1, 2)
    w0t = W0_w.T
    w2t = W2_w.T
    w0b = W0_b[None, :]
    w2b = W2_b[None, :]

    grid = (B // _S,)
    out_shape = (
        jax.ShapeDtypeStruct((B, _N, _E), jnp.float32),
        jax.ShapeDtypeStruct((B, 1, _E), jnp.float32),
    )
    Fv2, mean = pl.pallas_call(
        _ccn2_body,
        grid=grid,
        in_specs=[
            pl.BlockSpec((_S, _N, 3), lambda b: (b, 0, 0)),
            pl.BlockSpec((_S, 3, _N), lambda b: (b, 0, 0)),
            pl.BlockSpec((3, _E), lambda b: (0, 0)),
            pl.BlockSpec((1, _E), lambda b: (0, 0)),
            pl.BlockSpec((_E, _E), lambda b: (0, 0)),
            pl.BlockSpec((1, _E), lambda b: (0, 0)),
        ],
        out_specs=(
            pl.BlockSpec((_S, _N, _E), lambda b: (b, 0, 0)),
            pl.BlockSpec((_S, 1, _E), lambda b: (b, 0, 0)),
        ),
        out_shape=out_shape,
        compiler_params=pltpu.CompilerParams(
            dimension_semantics=("arbitrary",),
        ),
    )(feat, featT, w0t, w0b, w2t, w2b)
    del mean
    return Fv2, _sc_mean(Fv2)


# speedup vs baseline: 1.4950x; 1.4950x over previous
import functools

import jax
import jax.numpy as jnp
from jax.experimental import pallas as pl
from jax.experimental.pallas import tpu as pltpu

_THRESH = 0.055
_N = 500
_E = 128
_S = 16


def _one_sample(f, ft, w0t, w0b, w2t, w2b):
    xc = f[:, 0:1]
    yc = f[:, 1:2]
    xr = ft[0:1, :]
    yr = ft[1:2, :]
    dx = xc - xr
    dy = yc - yr
    dist2 = dx * dx + dy * dy
    A = (dist2 <= _THRESH * _THRESH).astype(jnp.float32)
    Ab = A.astype(jnp.bfloat16)

    fv0 = jnp.maximum(
        jnp.dot(f, w0t, preferred_element_type=jnp.float32) + w0b, 0.0)
    fh = fv0.astype(jnp.bfloat16)
    fl = (fv0 - fh.astype(jnp.float32)).astype(jnp.bfloat16)
    zcols = jnp.zeros((_N, 12), jnp.bfloat16)
    AF = jnp.concatenate([Ab, zcols, fh, fl], axis=1)
    R = jnp.dot(Ab, AF, preferred_element_type=jnp.float32)
    C = R[:, :_N]
    fv1 = R[:, 512:512 + _E] + R[:, 512 + _E:512 + 2 * _E]
    B2 = (C > 0).astype(jnp.float32)
    D = jnp.dot(B2.astype(jnp.bfloat16), Ab, preferred_element_type=jnp.float32)
    M = B2 * D

    g = jnp.dot(fv1, w2t, preferred_element_type=jnp.float32)
    Fv2 = jnp.maximum(
        jnp.dot(M, g, preferred_element_type=jnp.float32) + w2b, 0.0)
    return Fv2


def _ccn2_body(feat_ref, featT_ref, w0t_ref, w0b_ref, w2t_ref, w2b_ref,
               out_ref, mean_ref):
    for s in range(_S):
        Fv2 = _one_sample(feat_ref[s], featT_ref[s], w0t_ref[...],
                          w0b_ref[...], w2t_ref[...], w2b_ref[...])
        out_ref[s] = Fv2
        mean_ref[s, 0] = jnp.mean(Fv2, axis=0)


@functools.partial(jax.jit, static_argnames=())
def kernel(loc, deadline, depot, W0_w, W0_b, W2_w, W2_b):
    B = loc.shape[0]
    locations = jnp.concatenate([depot[:, None, :], loc], axis=1)
    td = jnp.concatenate(
        [jnp.zeros((B, 1), deadline.dtype), deadline], axis=1)
    feat = jnp.concatenate([locations, td[..., None]], axis=-1)
    featT = jnp.swapaxes(feat, 1, 2)
    w0t = W0_w.T
    w2t = W2_w.T
    w0b = W0_b[None, :]
    w2b = W2_b[None, :]

    grid = (B // _S,)
    out_shape = (
        jax.ShapeDtypeStruct((B, _N, _E), jnp.float32),
        jax.ShapeDtypeStruct((B, 1, _E), jnp.float32),
    )
    Fv2, mean = pl.pallas_call(
        _ccn2_body,
        grid=grid,
        in_specs=[
            pl.BlockSpec((_S, _N, 3), lambda b: (b, 0, 0)),
            pl.BlockSpec((_S, 3, _N), lambda b: (b, 0, 0)),
            pl.BlockSpec((3, _E), lambda b: (0, 0)),
            pl.BlockSpec((1, _E), lambda b: (0, 0)),
            pl.BlockSpec((_E, _E), lambda b: (0, 0)),
            pl.BlockSpec((1, _E), lambda b: (0, 0)),
        ],
        out_specs=(
            pl.BlockSpec((_S, _N, _E), lambda b: (b, 0, 0)),
            pl.BlockSpec((_S, 1, _E), lambda b: (b, 0, 0)),
        ),
        out_shape=out_shape,
        compiler_params=pltpu.CompilerParams(
            dimension_semantics=("arbitrary",),
        ),
    )(feat, featT, w0t, w0b, w2t, w2b)
    return Fv2, mean[:, 0, :]
